# baseline (speedup 1.0000x reference)
import functools

import jax
import jax.numpy as jnp
from jax import lax
from jax.experimental import pallas as pl
from jax.experimental.pallas import tpu as pltpu
from jax.experimental.pallas import tpu_sc as plsc

N_STEPS = 20
B = 1024
V = 100000
C = 4000
NJ = V // C
NCORE = 2
NSUB = 16
NW = NCORE * NSUB
BPW = B // NW
LANES = 16
MC = N_STEPS * C


def _body(probs_hbm, mask_hbm, step_hbm, out_hbm, step_v, maskc_v, prob_v):
    cid = lax.axis_index("c")
    sid = lax.axis_index("s")
    wid = sid * NCORE + cid
    gb0 = wid * BPW
    pltpu.sync_copy(step_hbm.at[pl.ds(gb0, BPW)], step_v)

    def chunk_body(j, _):
        pltpu.sync_copy(mask_hbm.at[pl.ds(j * MC, MC)], maskc_v)

        for bh in range(BPW // LANES):
            stepvec = step_v[pl.ds(bh * LANES, LANES)] - 1
            for lane in range(LANES):
                b = bh * LANES + lane
                rbase = stepvec[lane] * C
                pltpu.sync_copy(
                    probs_hbm.at[pl.ds((gb0 + b) * V + j * C, C)], prob_v)

                def vec_body(k, _, rbase=rbase):
                    m = maskc_v[pl.ds(rbase + k * LANES, LANES)]
                    p = prob_v[pl.ds(k * LANES, LANES)]
                    prob_v[pl.ds(k * LANES, LANES)] = jnp.where(
                        m != 0.0, p, 0.0)
                    return 0

                lax.fori_loop(0, C // LANES, vec_body, 0)
                pltpu.sync_copy(
                    prob_v, out_hbm.at[pl.ds((gb0 + b) * V + j * C, C)])
        return 0

    lax.fori_loop(0, NJ, chunk_body, 0)


def kernel(probabilites, mask, step):
    mask_t = mask.reshape(N_STEPS, NJ, C).transpose(1, 0, 2).reshape(-1)
    probs_flat = probabilites.reshape(-1)
    mesh = plsc.VectorSubcoreMesh(core_axis_name="c", subcore_axis_name="s")
    f = pl.kernel(
        _body,
        mesh=mesh,
        out_type=jax.ShapeDtypeStruct((B * V,), jnp.float32),
        scratch_types=[
            pltpu.VMEM((BPW,), jnp.int32),
            pltpu.VMEM((MC,), jnp.float32),
            pltpu.VMEM((C,), jnp.float32),
        ],
    )
    return f(probs_flat, mask_t, step).reshape(B, V)


# speedup vs baseline: 1.3212x; 1.3212x over previous
import functools

import jax
import jax.numpy as jnp
from jax import lax
from jax.experimental import pallas as pl
from jax.experimental.pallas import tpu as pltpu
from jax.experimental.pallas import tpu_sc as plsc

N_STEPS = 20
B = 1024
V = 100000
NCORE = 2
NSUB = 16
NW = NCORE * NSUB
LANES = 16
SW = 3128
STRIDE = 3136
NG = 196
UNROLL = 4
NIN = 8
NOUT = 4
NGB = B // LANES


def _body(probs_hbm, mask_hbm, step_hbm, out_hbm, *refs):
    step_v = refs[0]
    mask_v = refs[1]
    in_bufs = refs[2:2 + NIN]
    out_bufs = refs[2 + NIN:2 + NIN + NOUT]
    sem_stage = refs[2 + NIN + NOUT]
    in_sems = refs[3 + NIN + NOUT:3 + 2 * NIN + NOUT]
    out_sems = refs[3 + 2 * NIN + NOUT:3 + 2 * NIN + 2 * NOUT]

    cid = lax.axis_index("c")
    sid = lax.axis_index("s")
    wid = sid * NCORE + cid
    colbase = jnp.minimum(wid * SW, V - SW)

    pltpu.sync_copy(step_hbm, step_v)
    mask_copies = [
        pltpu.async_copy(
            mask_hbm.at[pl.ds(s * V + colbase, SW)],
            mask_v.at[pl.ds(s * STRIDE, SW)], sem_stage)
        for s in range(N_STEPS)
    ]
    for c in mask_copies:
        c.wait()

    def in_copy(b, i):
        return pltpu.async_copy(
            probs_hbm.at[pl.ds(b * V + colbase, SW)],
            in_bufs[i].at[pl.ds(0, SW)], in_sems[i])

    def out_copy(b, i):
        return pltpu.async_copy(
            out_bufs[i].at[pl.ds(0, SW)],
            out_hbm.at[pl.ds(b * V + colbase, SW)], out_sems[i])

    for b in range(NIN):
        in_copy(b, b)

    def group_body(g, _):
        stepvec = step_v[pl.ds(g * LANES, LANES)] - 1
        for lane in range(LANES):
            b = g * LANES + lane
            ib = lane % NIN
            ob = lane % NOUT
            if lane < NOUT:
                @pl.when(g > 0)
                def _():
                    pltpu.make_async_copy(
                        out_bufs[ob].at[pl.ds(0, SW)],
                        out_hbm.at[pl.ds(colbase, SW)], out_sems[ob]).wait()
            else:
                pltpu.make_async_copy(
                    out_bufs[ob].at[pl.ds(0, SW)],
                    out_hbm.at[pl.ds(colbase, SW)], out_sems[ob]).wait()
            pltpu.make_async_copy(
                probs_hbm.at[pl.ds(colbase, SW)],
                in_bufs[ib].at[pl.ds(0, SW)], in_sems[ib]).wait()

            rbase = stepvec[lane] * STRIDE
            ibuf = in_bufs[ib]
            obuf = out_bufs[ob]

            def vec_body(k, _, rbase=rbase, ibuf=ibuf, obuf=obuf):
                for u in range(UNROLL):
                    off = (k * UNROLL + u) * LANES
                    m = mask_v[pl.ds(rbase + off, LANES)]
                    p = ibuf[pl.ds(off, LANES)]
                    obuf[pl.ds(off, LANES)] = jnp.where(m != 0.0, p, 0.0)
                return 0

            lax.fori_loop(0, NG // UNROLL, vec_body, 0)
            out_copy(b, ob)
            if lane < NIN:
                in_copy(b + NIN, ib)
            else:
                @pl.when(g < NGB - 1)
                def _():
                    in_copy(b + NIN, ib)
        return 0

    lax.fori_loop(0, NGB, group_body, 0)

    for ob in range(NOUT):
        pltpu.make_async_copy(
            out_bufs[ob].at[pl.ds(0, SW)],
            out_hbm.at[pl.ds(colbase, SW)], out_sems[ob]).wait()


def kernel(probabilites, mask, step):
    probs_flat = probabilites.reshape(-1)
    mask_flat = mask.reshape(-1)
    mesh = plsc.VectorSubcoreMesh(core_axis_name="c", subcore_axis_name="s")
    scratch = [
        pltpu.VMEM((B,), jnp.int32),
        pltpu.VMEM((N_STEPS * STRIDE,), jnp.float32),
    ]
    scratch += [pltpu.VMEM((STRIDE,), jnp.float32) for _ in range(NIN)]
    scratch += [pltpu.VMEM((STRIDE,), jnp.float32) for _ in range(NOUT)]
    scratch += [pltpu.SemaphoreType.DMA] * (1 + NIN + NOUT)
    f = pl.kernel(
        _body,
        mesh=mesh,
        out_type=jax.ShapeDtypeStruct((B * V,), jnp.float32),
        scratch_types=scratch,
    )
    return f(probs_flat, mask_flat, step).reshape(B, V)


# speedup vs baseline: 1.8899x; 1.4304x over previous
import functools

import jax
import jax.numpy as jnp
from jax import lax
from jax.experimental import pallas as pl
from jax.experimental.pallas import tpu as pltpu
from jax.experimental.pallas import tpu_sc as plsc

N_STEPS = 20
B = 1024
V = 100000
NCORE = 2
NSUB = 16
NW = NCORE * NSUB
LANES = 16
SW = 3128
STRIDE = 3136
NG = 196
UNROLL = 14
NIN = 8
NOUT = 4
NGB = B // LANES


def _body(probs_hbm, mask_hbm, step_hbm, out_hbm, *refs):
    step_v = refs[0]
    mask_v = refs[1]
    in_bufs = refs[2:2 + NIN]
    out_bufs = refs[2 + NIN:2 + NIN + NOUT]
    sem_stage = refs[2 + NIN + NOUT]
    in_sems = refs[3 + NIN + NOUT:3 + 2 * NIN + NOUT]
    out_sems = refs[3 + 2 * NIN + NOUT:3 + 2 * NIN + 2 * NOUT]

    cid = lax.axis_index("c")
    sid = lax.axis_index("s")
    wid = sid * NCORE + cid
    colbase = jnp.minimum(wid * SW, V - SW)

    pltpu.sync_copy(step_hbm, step_v)
    mask_copies = [
        pltpu.async_copy(
            mask_hbm.at[pl.ds(s * V + colbase, SW)],
            mask_v.at[pl.ds(s * STRIDE, SW)], sem_stage)
        for s in range(N_STEPS)
    ]
    for c in mask_copies:
        c.wait()

    def in_copy(b, i):
        return pltpu.async_copy(
            probs_hbm.at[pl.ds(b * V + colbase, SW)],
            in_bufs[i].at[pl.ds(0, SW)], in_sems[i])

    def out_copy(b, i):
        return pltpu.async_copy(
            out_bufs[i].at[pl.ds(0, SW)],
            out_hbm.at[pl.ds(b * V + colbase, SW)], out_sems[i])

    for b in range(NIN):
        in_copy(b, b)

    def group_body(g, _):
        stepvec = step_v[pl.ds(g * LANES, LANES)] - 1
        for lane in range(LANES):
            b = g * LANES + lane
            ib = lane % NIN
            ob = lane % NOUT
            if lane < NOUT:
                @pl.when(g > 0)
                def _():
                    pltpu.make_async_copy(
                        out_bufs[ob].at[pl.ds(0, SW)],
                        out_hbm.at[pl.ds(colbase, SW)], out_sems[ob]).wait()
            else:
                pltpu.make_async_copy(
                    out_bufs[ob].at[pl.ds(0, SW)],
                    out_hbm.at[pl.ds(colbase, SW)], out_sems[ob]).wait()
            pltpu.make_async_copy(
                probs_hbm.at[pl.ds(colbase, SW)],
                in_bufs[ib].at[pl.ds(0, SW)], in_sems[ib]).wait()

            rbase = stepvec[lane] * STRIDE
            ibuf = in_bufs[ib]
            obuf = out_bufs[ob]

            @plsc.parallel_loop(0, NG, 1, unroll=UNROLL)
            def _(k, rbase=rbase, ibuf=ibuf, obuf=obuf):
                off = k * LANES
                m = mask_v[pl.ds(rbase + off, LANES)]
                obuf[pl.ds(off, LANES)] = m * ibuf[pl.ds(off, LANES)]
            out_copy(b, ob)
            if lane < NIN:
                in_copy(b + NIN, ib)
            else:
                @pl.when(g < NGB - 1)
                def _():
                    in_copy(b + NIN, ib)
        return 0

    lax.fori_loop(0, NGB, group_body, 0)

    for ob in range(NOUT):
        pltpu.make_async_copy(
            out_bufs[ob].at[pl.ds(0, SW)],
            out_hbm.at[pl.ds(colbase, SW)], out_sems[ob]).wait()


def kernel(probabilites, mask, step):
    probs_flat = probabilites.reshape(-1)
    mask_flat = mask.reshape(-1)
    mesh = plsc.VectorSubcoreMesh(core_axis_name="c", subcore_axis_name="s")
    scratch = [
        pltpu.VMEM((B,), jnp.int32),
        pltpu.VMEM((N_STEPS * STRIDE,), jnp.float32),
    ]
    scratch += [pltpu.VMEM((STRIDE,), jnp.float32) for _ in range(NIN)]
    scratch += [pltpu.VMEM((STRIDE,), jnp.float32) for _ in range(NOUT)]
    scratch += [pltpu.SemaphoreType.DMA] * (1 + NIN + NOUT)
    f = pl.kernel(
        _body,
        mesh=mesh,
        out_type=jax.ShapeDtypeStruct((B * V,), jnp.float32),
        scratch_types=scratch,
    )
    return f(probs_flat, mask_flat, step).reshape(B, V)


# speedup vs baseline: 3.8588x; 2.0418x over previous
import functools

import jax
import jax.numpy as jnp
from jax import lax
from jax.experimental import pallas as pl
from jax.experimental.pallas import tpu as pltpu
from jax.experimental.pallas import tpu_sc as plsc

N_STEPS = 20
B = 1024
V = 100000
NCORE = 2
NSUB = 16
NW = NCORE * NSUB
LANES = 16
TILE = 128
NTILES = V // TILE
NWIDE = NTILES - 24 * NW
WA = 12 * TILE
WC = TILE
SW = 25 * TILE
NRG = B // 8
MPAD = SW


def _body(probs_hbm, mask_hbm, step_hbm, out_hbm, *refs):
    (step_v, mask_v, in_a, out_a, in_b, out_b, in_c, out_c,
     sem_stage, si_a, so_a, si_b, so_b, si_c, so_c) = refs

    cid = lax.axis_index("c")
    sid = lax.axis_index("s")
    wid = sid * NCORE + cid
    c0 = (wid * 24 + jnp.minimum(wid, NWIDE)) * TILE
    is_wide = wid < NWIDE

    pltpu.sync_copy(step_hbm, step_v.at[pl.ds(0, B)])
    mask_copies = [
        pltpu.async_copy(
            mask_hbm.at[pl.ds(s * V + c0, SW)],
            mask_v.at[pl.ds(s * SW, SW)], sem_stage)
        for s in range(N_STEPS)
    ]
    for c in mask_copies:
        c.wait()

    def in_copy(rg, buf, sem, coff, w):
        pltpu.async_copy(
            probs_hbm.at[pl.ds(rg * 8, 8), pl.ds(coff, w)], buf, sem)

    def out_copy(rg, buf, sem, coff, w):
        pltpu.async_copy(
            buf, out_hbm.at[pl.ds(rg * 8, 8), pl.ds(coff, w)], sem)

    pieces = [
        (in_a, out_a, si_a, so_a, c0, WA, 0, False),
        (in_b, out_b, si_b, so_b, c0 + WA, WA, WA, False),
        (in_c, out_c, si_c, so_c, c0 + 2 * WA, WC, 2 * WA, True),
    ]

    def guarded(wide_only, fn):
        if wide_only:
            pl.when(is_wide)(fn)
        else:
            fn()

    for (ibuf, obuf, isem, osem, coff, w, soff, wide_only) in pieces:
        guarded(wide_only, functools.partial(in_copy, 0, ibuf, isem,
                                             coff, w))

    def rg_body(rg, _):
        stepvec = step_v[pl.ds(rg * 8, LANES)] - 1
        rbases = [stepvec[r] * SW for r in range(8)]

        for (ibuf, obuf, isem, osem, coff, w, soff, wide_only) in pieces:
            def piece(ibuf=ibuf, obuf=obuf, isem=isem, osem=osem,
                      coff=coff, w=w, soff=soff):
                pltpu.make_async_copy(
                    probs_hbm.at[pl.ds(0, 8), pl.ds(0, w)],
                    ibuf, isem).wait()

                @pl.when(rg > 0)
                def _():
                    pltpu.make_async_copy(
                        obuf, out_hbm.at[pl.ds(0, 8), pl.ds(0, w)],
                        osem).wait()

                @plsc.parallel_loop(0, w // LANES, 1,
                                    unroll=2 if w > WC else 1)
                def _(k):
                    off = k * LANES
                    for r in range(8):
                        m = mask_v[pl.ds(rbases[r] + soff + off, LANES)]
                        p = ibuf[r, pl.ds(off, LANES)]
                        obuf[r, pl.ds(off, LANES)] = m * p

                out_copy(rg, obuf, osem, coff, w)

                @pl.when(rg < NRG - 1)
                def _():
                    in_copy(rg + 1, ibuf, isem, coff, w)

            guarded(wide_only, piece)
        return 0

    lax.fori_loop(0, NRG, rg_body, 0)

    for (ibuf, obuf, isem, osem, coff, w, soff, wide_only) in pieces:
        guarded(wide_only, functools.partial(
            lambda ob, sm, ww: pltpu.make_async_copy(
                ob, out_hbm.at[pl.ds(0, 8), pl.ds(0, ww)], sm).wait(),
            obuf, osem, w))


def kernel(probabilites, mask, step):
    mask_flat = jnp.concatenate(
        [mask.reshape(-1), jnp.zeros((MPAD,), jnp.float32)])
    mesh = plsc.VectorSubcoreMesh(core_axis_name="c", subcore_axis_name="s")
    scratch = [
        pltpu.VMEM((B + LANES,), jnp.int32),
        pltpu.VMEM((N_STEPS * SW,), jnp.float32),
        pltpu.VMEM((8, WA), jnp.float32),
        pltpu.VMEM((8, WA), jnp.float32),
        pltpu.VMEM((8, WA), jnp.float32),
        pltpu.VMEM((8, WA), jnp.float32),
        pltpu.VMEM((8, WC), jnp.float32),
        pltpu.VMEM((8, WC), jnp.float32),
    ]
    scratch += [pltpu.SemaphoreType.DMA] * 7
    f = pl.kernel(
        _body,
        mesh=mesh,
        out_type=jax.ShapeDtypeStruct((B, V), jnp.float32),
        scratch_types=scratch,
    )
    out = f(probabilites, mask_flat, step)
    tail_c = NTILES * TILE
    tail_mask = jnp.take(mask[:, tail_c:], step - 1, axis=0)
    tail = probabilites[:, tail_c:] * tail_mask
    return lax.dynamic_update_slice(out, tail, (0, tail_c))


# speedup vs baseline: 12.5708x; 3.2577x over previous
import functools

import jax
import jax.numpy as jnp
from jax import lax
from jax.experimental import pallas as pl
from jax.experimental.pallas import tpu as pltpu
from jax.experimental.pallas import tpu_sc as plsc

N_STEPS = 20
B = 1024
V = 100000
NCORE = 2
NSUB = 16
NW = NCORE * NSUB
LANES = 16
SW = 3136
R = 16
NP = SW // R
NBG = B // LANES


def _body(probs_hbm, bits_hbm, step_hbm, out_hbm, *refs):
    (step_v, sh_v, bits_v, in_a, out_a, in_b, out_b,
     sem_stage, si_a, so_a, si_b, so_b) = refs

    cid = lax.axis_index("c")
    sid = lax.axis_index("s")
    wid = sid * NCORE + cid
    v0 = jnp.minimum(wid * SW, V - SW)

    pltpu.sync_copy(step_hbm, step_v)
    pltpu.sync_copy(bits_hbm.at[pl.ds(v0, SW)], bits_v)

    def sh_xform(g, _):
        sh_v[pl.ds(g * LANES, LANES)] = 32 - step_v[pl.ds(g * LANES, LANES)]
        return 0

    lax.fori_loop(0, NBG, sh_xform, 0)

    def in_copy(p, buf, sem):
        pltpu.async_copy(
            probs_hbm.at[pl.ds(v0 + p * R, R), pl.ds(0, B)], buf, sem)

    def out_copy(p, buf, sem):
        pltpu.async_copy(
            buf, out_hbm.at[pl.ds(v0 + p * R, R), pl.ds(0, B)], sem)

    halves = [(in_a, out_a, si_a, so_a), (in_b, out_b, si_b, so_b)]

    for h, (ibuf, obuf, isem, osem) in enumerate(halves):
        in_copy(h, ibuf, isem)

    def pair_body(pp, _):
        for h, (ibuf, obuf, isem, osem) in enumerate(halves):
            p = pp * 2 + h

            def half(ibuf=ibuf, obuf=obuf, isem=isem, osem=osem, p=p):
                pltpu.make_async_copy(
                    probs_hbm.at[pl.ds(0, R), pl.ds(0, B)],
                    ibuf, isem).wait()

                @pl.when(pp > 0)
                def _():
                    pltpu.make_async_copy(
                        obuf, out_hbm.at[pl.ds(0, R), pl.ds(0, B)],
                        osem).wait()

                mbvec = bits_v[pl.ds(p * R, R)]
                words = [jnp.broadcast_to(mbvec[r], (LANES,))
                         for r in range(R)]

                @plsc.parallel_loop(0, NBG, 1)
                def _(bg):
                    boff = bg * LANES
                    sh = sh_v[pl.ds(boff, LANES)]
                    for r in range(R):
                        t = words[r] << sh
                        pv = ibuf[r, pl.ds(boff, LANES)]
                        obuf[r, pl.ds(boff, LANES)] = jnp.where(
                            t < 0, pv, 0.0)

                out_copy(p, obuf, osem)

                @pl.when(pp < NP // 2 - 1)
                def _():
                    in_copy(p + 2, ibuf, isem)

            half()
        return 0

    lax.fori_loop(0, NP // 2, pair_body, 0)

    for h, (ibuf, obuf, isem, osem) in enumerate(halves):
        pltpu.make_async_copy(
            obuf, out_hbm.at[pl.ds(0, R), pl.ds(0, B)], osem).wait()


def kernel(probabilites, mask, step):
    bits = jnp.sum(
        jnp.where(mask != 0, 1, 0).astype(jnp.int32)
        << jnp.arange(N_STEPS, dtype=jnp.int32)[:, None],
        axis=0)
    probs_t = probabilites.T
    mesh = plsc.VectorSubcoreMesh(core_axis_name="c", subcore_axis_name="s")
    scratch = [
        pltpu.VMEM((B,), jnp.int32),
        pltpu.VMEM((B,), jnp.int32),
        pltpu.VMEM((SW,), jnp.int32),
        pltpu.VMEM((R, B), jnp.float32),
        pltpu.VMEM((R, B), jnp.float32),
        pltpu.VMEM((R, B), jnp.float32),
        pltpu.VMEM((R, B), jnp.float32),
    ]
    scratch += [pltpu.SemaphoreType.DMA] * 5
    f = pl.kernel(
        _body,
        mesh=mesh,
        out_type=jax.ShapeDtypeStruct((V, B), jnp.float32),
        scratch_types=scratch,
    )
    return f(probs_t, bits, step).T
